# Initial kernel scaffold; baseline (speedup 1.0000x reference)
import jax
import jax.numpy as jnp
from jax.experimental import pallas as pl


def kernel(x, W):
    raise NotImplementedError("write your pallas kernel here")


# speedup vs baseline: 1.5873x; 1.5873x over previous
import jax
import jax.numpy as jnp
from jax.experimental import pallas as pl

_N_CHAINS = 64
_N_STATES = 16
_CB = 8


def _gibbs_body(x_ref, w_ref, g_ref, o_ref):
    xv = x_ref[...]
    wv = w_ref[...]
    o_ref[...] = xv
    rowsum = jnp.sum(xv * wv, axis=1, keepdims=True)
    head = jnp.sum(xv[:, :_N_STATES] * wv[:, :_N_STATES], axis=1,
                   keepdims=True)
    logits = (rowsum - head) + wv[:, :_N_STATES] + g_ref[...]
    m = jnp.max(logits, axis=1, keepdims=True)
    iota = jax.lax.broadcasted_iota(jnp.int32, (_CB, _N_STATES), 1)
    sel = jnp.min(jnp.where(logits == m, iota, _N_STATES), axis=1,
                  keepdims=True)
    o_ref[:, :_N_STATES] = (iota == sel).astype(xv.dtype)


def kernel(x, W):
    n_chains, n_dims, n_states = x.shape
    flat = n_dims * n_states
    x2 = x.reshape(n_chains, flat)
    w2 = W.reshape(1, flat)
    g = jax.random.gumbel(jax.random.key(42), (n_chains, n_states),
                          dtype=x.dtype)
    out = pl.pallas_call(
        _gibbs_body,
        grid=(n_chains // _CB,),
        in_specs=[
            pl.BlockSpec((_CB, flat), lambda i: (i, 0)),
            pl.BlockSpec((1, flat), lambda i: (0, 0)),
            pl.BlockSpec((_CB, n_states), lambda i: (i, 0)),
        ],
        out_specs=pl.BlockSpec((_CB, flat), lambda i: (i, 0)),
        out_shape=jax.ShapeDtypeStruct((n_chains, flat), x.dtype),
    )(x2, w2, g)
    return out.reshape(n_chains, n_dims, n_states)
